# baseline (speedup 1.0000x reference)
import dataclasses
import functools

import jax
import jax.numpy as jnp
from jax import lax
from jax.experimental import pallas as pl
from jax.experimental.pallas import tpu as pltpu
from jax.experimental.pallas import tpu_sc as plsc

B = 16384
NUM_NUMERICAL = 410
H1 = 10
H2 = 12
OUT = 9
N_MOVE, D_MOVE, V_MOVE = 4, 16, 355
N_ABIL, D_ABIL, V_ABIL = 7, 8, 78
N_SLOTS = N_MOVE + N_ABIL
TP_ROWS = N_MOVE * V_MOVE + N_ABIL * V_ABIL
EMB_PAD = 128

_NC, _NS, _L = 2, 16, 16
_NW = _NC * _NS
_BPW = B // _NW
_NGRP = _BPW // _L


def _proj_body(mt_ref, at_ref, w1_ref, tp_ref):
    mt = mt_ref[...]
    at = at_ref[...]
    dn = (((0,), (1,)), ((), ()))
    parts = []
    for j in range(N_MOVE):
        off = NUM_NUMERICAL + D_MOVE * j
        parts.append(lax.dot_general(w1_ref[off:off + D_MOVE, :], mt, dn,
                                     preferred_element_type=jnp.float32))
    for j in range(N_ABIL):
        off = NUM_NUMERICAL + N_MOVE * D_MOVE + D_ABIL * j
        parts.append(lax.dot_general(w1_ref[off:off + D_ABIL, :], at, dn,
                                     preferred_element_type=jnp.float32))
    tp_ref[...] = jnp.concatenate(parts, axis=1)


def _project_tables(move_table, ability_table, W1):
    return pl.pallas_call(
        _proj_body,
        out_shape=jax.ShapeDtypeStruct((H1, TP_ROWS), jnp.float32),
    )(move_table, ability_table, W1)


def _sc_gather_body(tp_hbm, idx_hbm, out_hbm,
                    tp_v, idx_v, out_v, sem, sem2):
    core = lax.axis_index("c")
    sub = lax.axis_index("s")
    wid = sub * _NC + core
    c1 = pltpu.async_copy(tp_hbm, tp_v, sem)
    c2 = pltpu.async_copy(idx_hbm.at[pl.ds(wid * _BPW, _BPW), :], idx_v,
                          sem2)
    c1.wait()
    c2.wait()

    lane = lax.iota(jnp.int32, _L)
    col_ids = [jnp.full((_L,), c, jnp.int32) for c in range(H1)]
    slot_ids = [jnp.full((_L,), j, jnp.int32) for j in range(N_SLOTS)]

    @plsc.parallel_loop(0, _NGRP)
    def _(g):
        rows = g * _L + lane
        sidx = [plsc.load_gather(idx_v, [rows, slot_ids[j]])
                for j in range(N_SLOTS)]
        for c in range(H1):
            acc = plsc.load_gather(tp_v, [col_ids[c], sidx[0]])
            for j in range(1, N_SLOTS):
                acc = acc + plsc.load_gather(tp_v, [col_ids[c], sidx[j]])
            plsc.store_scatter(out_v, [rows, col_ids[c]], acc)

    pltpu.async_copy(out_v, out_hbm.at[pl.ds(wid * _BPW, _BPW), :],
                     sem).wait()


def _sc_gather(tp_t, idx2):
    mesh = plsc.VectorSubcoreMesh(core_axis_name="c", subcore_axis_name="s",
                                  num_cores=_NC, num_subcores=_NS)
    cp = pltpu.CompilerParams()
    fields = pltpu.CompilerParams.__dataclass_fields__
    if "needs_layout_passes" in fields:
        cp = dataclasses.replace(cp, needs_layout_passes=False)
    if "use_tc_tiling_on_sc" in fields:
        cp = dataclasses.replace(cp, use_tc_tiling_on_sc=False)
    k = pl.kernel(
        _sc_gather_body,
        out_type=jax.ShapeDtypeStruct((B, EMB_PAD), jnp.float32),
        mesh=mesh,
        compiler_params=cp,
        scratch_types=[
            pltpu.VMEM((H1, TP_ROWS), jnp.float32),
            pltpu.VMEM((_BPW, N_SLOTS), jnp.int32),
            pltpu.VMEM((_BPW, EMB_PAD), jnp.float32),
            pltpu.SemaphoreType.DMA,
            pltpu.SemaphoreType.DMA,
        ],
    )
    return k(tp_t, idx2)


_BLK = 4096


def _mega_body(x_ref, e_ref, w1_ref, b1_ref, w2_ref, b2_ref, w3_ref, b3_ref,
               o_ref):
    p = jnp.dot(x_ref[...], w1_ref[...], preferred_element_type=jnp.float32)
    h1 = jnp.maximum(p + e_ref[...][:, :H1] + b1_ref[...], 0.0)
    h2 = jnp.dot(h1, w2_ref[...], preferred_element_type=jnp.float32)
    h2 = jnp.maximum(h2 + b2_ref[...], 0.0)
    o_ref[...] = jnp.dot(h2, w3_ref[...],
                         preferred_element_type=jnp.float32) + b3_ref[...]


def _mega(x, emb, W1n, b1, W2, b2, W3, b3):
    return pl.pallas_call(
        _mega_body,
        grid=(B // _BLK,),
        in_specs=[
            pl.BlockSpec((_BLK, NUM_NUMERICAL), lambda i: (i, 0)),
            pl.BlockSpec((_BLK, EMB_PAD), lambda i: (i, 0)),
            pl.BlockSpec((NUM_NUMERICAL, H1), lambda i: (0, 0)),
            pl.BlockSpec((1, H1), lambda i: (0, 0)),
            pl.BlockSpec((H1, H2), lambda i: (0, 0)),
            pl.BlockSpec((1, H2), lambda i: (0, 0)),
            pl.BlockSpec((H2, OUT), lambda i: (0, 0)),
            pl.BlockSpec((1, OUT), lambda i: (0, 0)),
        ],
        out_specs=pl.BlockSpec((_BLK, OUT), lambda i: (i, 0)),
        out_shape=jax.ShapeDtypeStruct((B, OUT), jnp.float32),
    )(x, emb, W1n, b1.reshape(1, H1), W2, b2.reshape(1, H2), W3,
      b3.reshape(1, OUT))


def kernel(x_numerical_tensor, move_effect_tensor, ability_tensor,
           move_table, ability_table, W1, b1, W2, b2, W3, b3):
    move_off = jnp.arange(N_MOVE, dtype=jnp.int32) * V_MOVE
    abil_off = (N_MOVE * V_MOVE
                + jnp.arange(N_ABIL, dtype=jnp.int32) * V_ABIL)
    flat = jnp.concatenate([
        move_effect_tensor.astype(jnp.int32) + move_off[None, :],
        ability_tensor.astype(jnp.int32) + abil_off[None, :],
    ], axis=1)

    tp_t = _project_tables(move_table, ability_table, W1)
    emb = _sc_gather(tp_t, flat)
    return _mega(x_numerical_tensor, emb, W1[:NUM_NUMERICAL, :],
                 b1, W2, b2, W3, b3)


# speedup vs baseline: 1.0502x; 1.0502x over previous
import dataclasses
import functools

import jax
import jax.numpy as jnp
from jax import lax
from jax.experimental import pallas as pl
from jax.experimental.pallas import tpu as pltpu
from jax.experimental.pallas import tpu_sc as plsc

B = 16384
NUM_NUMERICAL = 410
H1 = 10
H2 = 12
OUT = 9
N_MOVE, D_MOVE, V_MOVE = 4, 16, 355
N_ABIL, D_ABIL, V_ABIL = 7, 8, 78
N_SLOTS = N_MOVE + N_ABIL
TP_ROWS = N_MOVE * V_MOVE + N_ABIL * V_ABIL
EMB_PAD = 128

_NC, _NS, _L = 2, 16, 16
_NW = _NC * _NS
_BPW = B // _NW
_NGRP = _BPW // _L


def _proj_body(mt_ref, at_ref, w1_ref, tp_ref):
    mt = mt_ref[...]
    at = at_ref[...]
    dn = (((0,), (1,)), ((), ()))
    parts = []
    for j in range(N_MOVE):
        off = NUM_NUMERICAL + D_MOVE * j
        parts.append(lax.dot_general(w1_ref[off:off + D_MOVE, :], mt, dn,
                                     preferred_element_type=jnp.float32))
    for j in range(N_ABIL):
        off = NUM_NUMERICAL + N_MOVE * D_MOVE + D_ABIL * j
        parts.append(lax.dot_general(w1_ref[off:off + D_ABIL, :], at, dn,
                                     preferred_element_type=jnp.float32))
    tp_ref[...] = jnp.concatenate(parts, axis=1)


def _project_tables(move_table, ability_table, W1):
    return pl.pallas_call(
        _proj_body,
        out_shape=jax.ShapeDtypeStruct((H1, TP_ROWS), jnp.float32),
    )(move_table, ability_table, W1)


def _sc_gather_body(tp_hbm, idx_hbm, out_hbm,
                    tp_v, idx_v, out_v, sem, sem2):
    core = lax.axis_index("c")
    sub = lax.axis_index("s")
    wid = sub * _NC + core
    c1 = pltpu.async_copy(tp_hbm, tp_v, sem)
    c2 = pltpu.async_copy(idx_hbm.at[wid], idx_v, sem2)
    c1.wait()
    c2.wait()

    lane = lax.iota(jnp.int32, _L)
    col_ids = [jnp.full((_L,), c, jnp.int32) for c in range(H1)]

    @plsc.parallel_loop(0, _NGRP)
    def _(g):
        rows = g * _L + lane
        pos = rows * N_SLOTS
        sidx = [plsc.load_gather(idx_v, [pos + j]) for j in range(N_SLOTS)]
        for c in range(H1):
            acc = plsc.load_gather(tp_v, [col_ids[c], sidx[0]])
            for j in range(1, N_SLOTS):
                acc = acc + plsc.load_gather(tp_v, [col_ids[c], sidx[j]])
            plsc.store_scatter(out_v, [rows, col_ids[c]], acc)

    pltpu.async_copy(out_v, out_hbm.at[pl.ds(wid * _BPW, _BPW), :],
                     sem).wait()


def _sc_gather(tp_t, idx2):
    mesh = plsc.VectorSubcoreMesh(core_axis_name="c", subcore_axis_name="s",
                                  num_cores=_NC, num_subcores=_NS)
    cp = pltpu.CompilerParams()
    fields = pltpu.CompilerParams.__dataclass_fields__
    if "needs_layout_passes" in fields:
        cp = dataclasses.replace(cp, needs_layout_passes=False)
    if "use_tc_tiling_on_sc" in fields:
        cp = dataclasses.replace(cp, use_tc_tiling_on_sc=False)
    k = pl.kernel(
        _sc_gather_body,
        out_type=jax.ShapeDtypeStruct((B, EMB_PAD), jnp.float32),
        mesh=mesh,
        compiler_params=cp,
        scratch_types=[
            pltpu.VMEM((H1, TP_ROWS), jnp.float32),
            pltpu.VMEM((_BPW * N_SLOTS,), jnp.int32),
            pltpu.VMEM((_BPW, EMB_PAD), jnp.float32),
            pltpu.SemaphoreType.DMA,
            pltpu.SemaphoreType.DMA,
        ],
    )
    return k(tp_t, idx2)


_BLK = 4096


def _mega_body(x_ref, e_ref, w1_ref, b1_ref, w2_ref, b2_ref, w3_ref, b3_ref,
               o_ref):
    p = jnp.dot(x_ref[...], w1_ref[...], preferred_element_type=jnp.float32)
    h1 = jnp.maximum(p + e_ref[...][:, :H1] + b1_ref[...], 0.0)
    h2 = jnp.dot(h1, w2_ref[...], preferred_element_type=jnp.float32)
    h2 = jnp.maximum(h2 + b2_ref[...], 0.0)
    o_ref[...] = jnp.dot(h2, w3_ref[...],
                         preferred_element_type=jnp.float32) + b3_ref[...]


def _mega(x, emb, W1n, b1, W2, b2, W3, b3):
    return pl.pallas_call(
        _mega_body,
        grid=(B // _BLK,),
        in_specs=[
            pl.BlockSpec((_BLK, NUM_NUMERICAL), lambda i: (i, 0)),
            pl.BlockSpec((_BLK, EMB_PAD), lambda i: (i, 0)),
            pl.BlockSpec((NUM_NUMERICAL, H1), lambda i: (0, 0)),
            pl.BlockSpec((1, H1), lambda i: (0, 0)),
            pl.BlockSpec((H1, H2), lambda i: (0, 0)),
            pl.BlockSpec((1, H2), lambda i: (0, 0)),
            pl.BlockSpec((H2, OUT), lambda i: (0, 0)),
            pl.BlockSpec((1, OUT), lambda i: (0, 0)),
        ],
        out_specs=pl.BlockSpec((_BLK, OUT), lambda i: (i, 0)),
        out_shape=jax.ShapeDtypeStruct((B, OUT), jnp.float32),
    )(x, emb, W1n, b1.reshape(1, H1), W2, b2.reshape(1, H2), W3,
      b3.reshape(1, OUT))


def kernel(x_numerical_tensor, move_effect_tensor, ability_tensor,
           move_table, ability_table, W1, b1, W2, b2, W3, b3):
    move_off = jnp.arange(N_MOVE, dtype=jnp.int32) * V_MOVE
    abil_off = (N_MOVE * V_MOVE
                + jnp.arange(N_ABIL, dtype=jnp.int32) * V_ABIL)
    flat = jnp.concatenate([
        move_effect_tensor.astype(jnp.int32) + move_off[None, :],
        ability_tensor.astype(jnp.int32) + abil_off[None, :],
    ], axis=1)
    idx2 = flat.reshape(_NW, _BPW * N_SLOTS)

    tp_t = _project_tables(move_table, ability_table, W1)
    emb = _sc_gather(tp_t, idx2)
    return _mega(x_numerical_tensor, emb, W1[:NUM_NUMERICAL, :],
                 b1, W2, b2, W3, b3)
